# baseline (speedup 1.0000x reference)
import jax
import jax.numpy as jnp
from jax.experimental import pallas as pl

_TB = 128
_LANES = 128
_ROWS_PER_TRIP = 125
_RB = _TB * _ROWS_PER_TRIP


def _body(x_ref, c_ref, o_ref):
    y = x_ref[...] * c_ref[...]
    lane = jax.lax.broadcasted_iota(jnp.int32, (_LANES, 8), 0)
    grp = jax.lax.broadcasted_iota(jnp.int32, (_LANES, 8), 1)
    g = (lane // 16 == grp).astype(jnp.float32)
    z = jax.lax.dot_general(y, g, (((1,), (0,)), ((), ())),
                            preferred_element_type=jnp.float32)
    o_ref[...] = z


def kernel(x, coef):
    num_trips, num_items, num_params = x.shape
    x2 = x.reshape(num_trips * num_items * num_params // _LANES, _LANES)
    ctile = jnp.tile(coef.reshape(1, num_items * num_params), (_TB, 1))
    ctile = ctile.reshape(_RB, _LANES)
    grid = (num_trips // _TB,)
    o2 = pl.pallas_call(
        _body,
        grid=grid,
        in_specs=[
            pl.BlockSpec((_RB, _LANES), lambda i: (i, 0)),
            pl.BlockSpec((_RB, _LANES), lambda i: (0, 0)),
        ],
        out_specs=pl.BlockSpec((_RB, 8), lambda i: (i, 0)),
        out_shape=jax.ShapeDtypeStruct((num_trips * num_items // 8, 8), jnp.float32),
    )(x2, ctile)
    return o2.reshape(num_trips, num_items)


# speedup vs baseline: 12.9538x; 12.9538x over previous
import jax
import jax.numpy as jnp
from jax.experimental import pallas as pl

_IB = 8


def _body(x_ref, c_ref, o_ref):
    o_ref[...] = jnp.sum(x_ref[...] * c_ref[...][:, :, None], axis=1)


def kernel(x, coef):
    num_trips, num_items, num_params = x.shape
    xt = jnp.transpose(x, (1, 2, 0))
    outT = pl.pallas_call(
        _body,
        grid=(num_items // _IB,),
        in_specs=[
            pl.BlockSpec((_IB, num_params, num_trips), lambda i: (i, 0, 0)),
            pl.BlockSpec((_IB, num_params), lambda i: (i, 0)),
        ],
        out_specs=pl.BlockSpec((_IB, num_trips), lambda i: (i, 0)),
        out_shape=jax.ShapeDtypeStruct((num_items, num_trips), jnp.float32),
    )(xt, coef)
    return outT.T


# speedup vs baseline: 20.5249x; 1.5845x over previous
import jax
import jax.numpy as jnp
from jax.experimental import pallas as pl

_IB = 40


def _body(x_ref, c_ref, o_ref):
    o_ref[...] = jnp.sum(x_ref[...] * c_ref[...][:, :, None], axis=1)


def kernel(x, coef):
    num_trips, num_items, num_params = x.shape
    xt = jnp.transpose(x, (1, 2, 0))
    outT = pl.pallas_call(
        _body,
        grid=(num_items // _IB,),
        in_specs=[
            pl.BlockSpec((_IB, num_params, num_trips), lambda i: (i, 0, 0)),
            pl.BlockSpec((_IB, num_params), lambda i: (i, 0)),
        ],
        out_specs=pl.BlockSpec((_IB, num_trips), lambda i: (i, 0)),
        out_shape=jax.ShapeDtypeStruct((num_items, num_trips), jnp.float32),
    )(xt, coef)
    return outT.T


# speedup vs baseline: 20.5421x; 1.0008x over previous
import jax
import jax.numpy as jnp
from jax.experimental import pallas as pl

_IB = 64


def _body(x_ref, c_ref, o_ref):
    o_ref[...] = jnp.sum(x_ref[...] * c_ref[...][:, :, None], axis=1)


def kernel(x, coef):
    num_trips, num_items, num_params = x.shape
    xt = jnp.transpose(x, (1, 2, 0))
    outT = pl.pallas_call(
        _body,
        grid=(pl.cdiv(num_items, _IB),),
        in_specs=[
            pl.BlockSpec((_IB, num_params, num_trips), lambda i: (i, 0, 0)),
            pl.BlockSpec((_IB, num_params), lambda i: (i, 0)),
        ],
        out_specs=pl.BlockSpec((_IB, num_trips), lambda i: (i, 0)),
        out_shape=jax.ShapeDtypeStruct((num_items, num_trips), jnp.float32),
    )(xt, coef)
    return outT.T
